# baseline (speedup 1.0000x reference)
import functools

import jax
import jax.numpy as jnp
from jax import lax
from jax.experimental import pallas as pl
from jax.experimental.pallas import tpu as pltpu
from jax.experimental.pallas import tpu_sc as plsc

N = 200
H = 7
C = 300
K = 20
NP = 256
HI = jax.lax.Precision.HIGHEST
NEG = -1e30

_NC, _NS, _L = 2, 16, 16
_NW = _NC * _NS
_ROWS_PER = 7
_CH = NP // _L
_CHR = 13


def _tc_a(x_ref, wggl_ref, bggl_ref, a_ref):
    z = jnp.dot(x_ref[...], wggl_ref[...], precision=HI) + bggl_ref[...]
    atrr = 1.0 / (1.0 + jnp.exp(-z))
    A = jax.lax.dot_general(atrr, atrr, (((1,), (1,)), ((), ())), precision=HI)
    a_ref[...] = jnp.concatenate(
        [A, jnp.full((N, NP - N), NEG, jnp.float32)], axis=1)


def _tc_hx(x_ref, wgatx_ref, hx_ref):
    hx_ref[...] = jnp.dot(x_ref[...], wgatx_ref[...])


def _sc_topk(a_hbm, adj_hbm, row_v, adj_v, scf_v, sci_v):
    wid = lax.axis_index("s") * _NC + lax.axis_index("c")
    lane = lax.iota(jnp.int32, _L)
    perms = [lane ^ s for s in (8, 4, 2, 1)]

    def _allmax(x):
        for p in perms:
            scf_v[...] = x
            x = jnp.maximum(x, plsc.load_gather(scf_v, [p]))
        return x

    def _allmin_i32(x):
        for p in perms:
            sci_v[...] = x
            x = jnp.minimum(x, plsc.load_gather(sci_v, [p]))
        return x

    def do_row(r, carry):
        row = wid * _ROWS_PER + r

        @pl.when(row < N)
        def _():
            pltpu.sync_copy(a_hbm.at[row], row_v)
            chunks = [row_v[pl.ds(c * _L, _L)] for c in range(_CHR)]
            iotas = [lane + c * _L for c in range(_CHR)]
            sel = [jnp.zeros((_L,), jnp.float32) for _ in range(_CH)]
            for _ in range(K):
                m = chunks[0]
                for c in range(1, _CHR):
                    m = jnp.maximum(m, chunks[c])
                mx = _allmax(m)
                cand = jnp.where(chunks[0] == mx, iotas[0], NP)
                for c in range(1, _CHR):
                    cand = jnp.minimum(
                        cand, jnp.where(chunks[c] == mx, iotas[c], NP))
                jstar = _allmin_i32(cand)
                for c in range(_CHR):
                    is_sel = iotas[c] == jstar
                    sel[c] = jnp.where(is_sel, 1.0, sel[c])
                    chunks[c] = jnp.where(is_sel, NEG, chunks[c])
            for c in range(_CH):
                adj_v[pl.ds(c * _L, _L)] = sel[c]
            pltpu.sync_copy(adj_v, adj_hbm.at[row])

        return carry

    lax.fori_loop(0, _ROWS_PER, do_row, 0)


def _sc_topk_call(a_pad):
    mesh = plsc.VectorSubcoreMesh(core_axis_name="c", subcore_axis_name="s",
                                  num_cores=_NC, num_subcores=_NS)
    return pl.kernel(
        _sc_topk,
        out_type=jax.ShapeDtypeStruct((N, NP), jnp.float32),
        mesh=mesh,
        scratch_types=[pltpu.VMEM((NP,), jnp.float32),
                       pltpu.VMEM((NP,), jnp.float32),
                       pltpu.VMEM((_L,), jnp.float32),
                       pltpu.VMEM((_L,), jnp.int32)],
        compiler_params=pltpu.CompilerParams(needs_layout_passes=False),
    )(a_pad)


def _tc_main(adj_ref, hx_ref, ein_ref, eout_ref, wgatio_ref,
             asad_ref, bgat_ref, gamma_ref, beta_ref, w3_ref, b3_ref,
             out_ref):
    f32 = jnp.float32
    adj = adj_ref[:, :N]

    row_i = jax.lax.broadcasted_iota(jnp.int32, (N, N), 0)
    col_j = jax.lax.broadcasted_iota(jnp.int32, (N, N), 1)

    ones_col = jnp.ones((N, 1), f32)
    in_deg = jax.lax.dot_general(adj, ones_col, (((0,), (0,)), ((), ())),
                                 precision=HI)
    in_idx = jnp.minimum(in_deg, float(N - 1))
    onehot_in = (col_j.astype(f32) == in_idx).astype(f32)
    in_f = jnp.dot(onehot_in, ein_ref[...], precision=HI)
    onehot_out = (col_j[:1, :] == K).astype(f32)
    orow = jnp.dot(onehot_out, eout_ref[...], precision=HI)

    h = (hx_ref[...] + jnp.dot(in_f, wgatio_ref[:8, :])
         + jnp.dot(orow, wgatio_ref[8:, :]))
    t2 = jax.lax.dot_general(asad_ref[...], h, (((0,), (1,)), ((), ())))
    es = jnp.transpose(t2[:H, :])
    ed_t = t2[H:, :]

    eye_f = (row_i == col_j).astype(f32)
    dist0 = 2.0 * eye_f - 1.0

    iota_col = jax.lax.broadcasted_iota(jnp.int32, (N, 1), 0)

    def bfs_cond(carry):
        return carry[3] != 0

    def _hop(d, dist, frontier):
        allowed = frontier * jnp.where(d < iota_col, 1.0, 0.0)
        reach = jnp.dot(allowed, adj)
        nxt = jnp.where((reach > 0.0) & (dist == -1.0), 1.0, 0.0)
        dist = jnp.where(nxt > 0.0, (d + 1).astype(f32), dist)
        return dist, nxt

    def bfs_body(carry):
        d, dist, frontier, _ = carry
        dist, nxt = _hop(d, dist, frontier)
        dist, nxt = _hop(d + 1, dist, nxt)
        go = jnp.where(jnp.any(nxt > 0.0), jnp.int32(1), jnp.int32(0))
        return d + 2, dist, nxt, go

    _, dist, _, _ = jax.lax.while_loop(
        bfs_cond, bfs_body,
        (jnp.int32(0), dist0, eye_f, jnp.int32(1)))
    bias = jnp.where(dist != -1.0, dist, NEG)

    outs = []
    for hh in range(H):
        v = es[:, hh:hh + 1] + ed_t[hh:hh + 1, :]
        logit = jnp.maximum(v, 0.2 * v) + bias
        m = jnp.max(logit, axis=0, keepdims=True)
        e = jnp.exp(logit - m)
        den = jnp.sum(e, axis=0, keepdims=True)
        alpha = e * (1.0 / (den + 1e-16))
        hcol = h[:, hh * C:(hh + 1) * C]
        outs.append(jax.lax.dot_general(
            alpha, hcol, (((0,), (0,)), ((), ()))))
    out = jnp.concatenate(outs, axis=1) + bgat_ref[...]

    mu = jnp.mean(out, axis=0, keepdims=True)
    ctr = out - mu
    var = jnp.mean(ctr * ctr, axis=0, keepdims=True)
    out = ctr * jax.lax.rsqrt(var + 1e-5) * gamma_ref[...] + beta_ref[...]

    res = jnp.dot(out, w3_ref[...]) + b3_ref[...]
    out_ref[...] = jnp.maximum(res, 0.0)


def kernel(x, W_ggl, b_ggl, emb_in, emb_out, W_gat, a_src, a_dst, b_gat,
           gamma, beta, W3, b3):
    eyeH = jnp.eye(H, dtype=jnp.float32)
    as_mat = (a_src[:, :, None] * eyeH[:, None, :]).reshape(H * C, H)
    ad_mat = (a_dst[:, :, None] * eyeH[:, None, :]).reshape(H * C, H)
    asad_mat = jnp.concatenate([as_mat, ad_mat], axis=1)

    a_pad = pl.pallas_call(
        _tc_a, out_shape=jax.ShapeDtypeStruct((N, NP), jnp.float32),
    )(x, W_ggl, b_ggl.reshape(1, -1))

    adj = _sc_topk_call(a_pad)

    h_x = pl.pallas_call(
        _tc_hx, out_shape=jax.ShapeDtypeStruct((N, H * C), jnp.float32),
    )(x, W_gat[:256, :])

    return pl.pallas_call(
        _tc_main, out_shape=jax.ShapeDtypeStruct((N, 256), jnp.float32),
    )(adj, h_x, emb_in, emb_out, W_gat[256:, :], asad_mat,
      b_gat.reshape(1, -1), gamma.reshape(1, -1), beta.reshape(1, -1),
      W3, b3.reshape(1, -1))


# speedup vs baseline: 2.1951x; 2.1951x over previous
import jax
import jax.numpy as jnp
from jax.experimental import pallas as pl

N = 200
H = 7
C = 300
K = 20
HI = jax.lax.Precision.HIGHEST
NEG = -1e30


def _fused(x_ref, wggl_ref, bggl_ref, ein_ref, eout_ref, wgat_ref,
           asad_ref, bgat_ref, gamma_ref, beta_ref, w3_ref, b3_ref,
           out_ref):
    f32 = jnp.float32
    x = x_ref[...]

    z = jnp.dot(x, wggl_ref[...], precision=HI) + bggl_ref[...]
    atrr = 1.0 / (1.0 + jnp.exp(-z))
    A = jax.lax.dot_general(atrr, atrr, (((1,), (1,)), ((), ())), precision=HI)

    row_i = jax.lax.broadcasted_iota(jnp.int32, (N, N), 0)
    col_j = jax.lax.broadcasted_iota(jnp.int32, (N, N), 1)

    a_work = A
    for _ in range(K):
        colmax = jnp.max(a_work, axis=0, keepdims=True)
        cand = jnp.where(a_work == colmax, row_i, N)
        jstar = jnp.min(cand, axis=0, keepdims=True)
        a_work = jnp.where(row_i == jstar, NEG, a_work)
    adjt = jnp.where(a_work == NEG, 1.0, 0.0)

    ones_col = jnp.ones((N, 1), f32)
    in_deg = jnp.dot(adjt, ones_col, precision=HI)
    in_idx = jnp.minimum(in_deg, float(N - 1))
    onehot_in = (col_j.astype(f32) == in_idx).astype(f32)
    in_f = jnp.dot(onehot_in, ein_ref[...], precision=HI)
    onehot_out = (col_j[:1, :] == K).astype(f32)
    orow = jnp.dot(onehot_out, eout_ref[...], precision=HI)
    out_f = jnp.broadcast_to(orow, (N, 8))

    in_cat = jnp.concatenate([x, in_f, out_f], axis=1)
    h = jnp.dot(in_cat, wgat_ref[...])
    t2 = jax.lax.dot_general(asad_ref[...], h, (((0,), (1,)), ((), ())))
    es = jnp.transpose(t2[:H, :])
    ed_t = t2[H:, :]

    eye_f = (row_i == col_j).astype(f32)
    dist0 = 2.0 * eye_f - 1.0

    iota_col = jax.lax.broadcasted_iota(jnp.int32, (N, 1), 0)

    def bfs_cond(carry):
        return carry[3] != 0

    def _hop(d, dist, frontier):
        allowed = frontier * jnp.where(d < iota_col, 1.0, 0.0)
        reach = jax.lax.dot_general(allowed, adjt, (((1,), (1,)), ((), ())))
        nxt = jnp.where((reach > 0.0) & (dist == -1.0), 1.0, 0.0)
        dist = jnp.where(nxt > 0.0, (d + 1).astype(f32), dist)
        return dist, nxt

    def bfs_body(carry):
        d, dist, frontier, _ = carry
        dist, nxt = _hop(d, dist, frontier)
        dist, nxt = _hop(d + 1, dist, nxt)
        go = jnp.where(jnp.any(nxt > 0.0), jnp.int32(1), jnp.int32(0))
        return d + 2, dist, nxt, go

    dist, nxt = _hop(jnp.int32(0), dist0, eye_f)
    for dd in range(1, 4):
        dist, nxt = _hop(jnp.int32(dd), dist, nxt)
    go0 = jnp.where(jnp.any(nxt > 0.0), jnp.int32(1), jnp.int32(0))
    _, dist, _, _ = jax.lax.while_loop(
        bfs_cond, bfs_body,
        (jnp.int32(4), dist, nxt, go0))
    bias = jnp.where(dist != -1.0, dist, NEG)

    outs = []
    for hh in range(H):
        es_col = es[:, hh:hh + 1]
        ed_row = ed_t[hh:hh + 1, :]
        v = es_col + ed_row
        logit = jnp.maximum(v, 0.2 * v) + bias
        m = jnp.max(logit, axis=0, keepdims=True)
        e = jnp.exp(logit - m)
        den = jnp.sum(e, axis=0, keepdims=True)
        alpha = e * (1.0 / (den + 1e-16))
        hcol = h[:, hh * C:(hh + 1) * C]
        outs.append(jax.lax.dot_general(
            alpha, hcol, (((0,), (0,)), ((), ()))))
    out = jnp.concatenate(outs, axis=1) + bgat_ref[...]

    mu = jnp.mean(out, axis=0, keepdims=True)
    ctr = out - mu
    var = jnp.mean(ctr * ctr, axis=0, keepdims=True)
    out = ctr * jax.lax.rsqrt(var + 1e-5) * gamma_ref[...] + beta_ref[...]

    res = jnp.dot(out, w3_ref[...]) + b3_ref[...]
    out_ref[...] = jnp.maximum(res, 0.0)


def kernel(x, W_ggl, b_ggl, emb_in, emb_out, W_gat, a_src, a_dst, b_gat,
           gamma, beta, W3, b3):
    eyeH = jnp.eye(H, dtype=jnp.float32)
    as_mat = (a_src[:, :, None] * eyeH[:, None, :]).reshape(H * C, H)
    ad_mat = (a_dst[:, :, None] * eyeH[:, None, :]).reshape(H * C, H)
    asad_mat = jnp.concatenate([as_mat, ad_mat], axis=1)
    return pl.pallas_call(
        _fused,
        out_shape=jax.ShapeDtypeStruct((N, 256), jnp.float32),
    )(x, W_ggl, b_ggl.reshape(1, -1), emb_in, emb_out, W_gat,
      asad_mat, b_gat.reshape(1, -1), gamma.reshape(1, -1),
      beta.reshape(1, -1), W3, b3.reshape(1, -1))
